# Initial kernel scaffold; baseline (speedup 1.0000x reference)
import jax
import jax.numpy as jnp
from jax.experimental import pallas as pl


def kernel(nodes_feat, edges_feat, nodes_num_norm_sqrt, edges_num_norm_sqrt, W_emb, b_emb, Wfc0, Wa0, Wfc1, Wa1, Wfc2, Wa2, Wfc3, Wa3, Wr0, br0, Wr1, br1, Wr2, br2, edge_index):
    raise NotImplementedError("write your pallas kernel here")


# speedup vs baseline: 72.3721x; 72.3721x over previous
import functools

import jax
import jax.numpy as jnp
import numpy as np
from jax import lax
from jax.experimental import pallas as pl
from jax.experimental.pallas import tpu as pltpu
from jax.experimental.pallas import tpu_sc as plsc

N = 10000
E = 320000
F = 128
NH = 8
DH = 16
ZS_W = F + 16

NC, NS, LANES = 2, 16, 16
EPW = E // (NC * NS)
K = 200
NCHUNK = EPW // K
N_PAD = 10112
RPT = N_PAD // NS

_PERM = np.array([(l % 8) * 16 + 2 * k + l // 8
                  for k in range(8) for l in range(16)], dtype=np.int32)

_RDEN = np.zeros((16, F), dtype=np.float32)
for _l in range(8):
    _RDEN[_l, np.arange(F) % 8 == _l] = 1.0


def _prep_layer(Wfc, Wa):
    H = Wfc.shape[0]
    if H == NH:
        W2 = jnp.transpose(Wfc, (1, 0, 2)).reshape(F, F)
        h_of_d = np.arange(F) // DH
        mask = (h_of_d[:, None] == (np.arange(16)[None, :] % 8)).astype(np.float32)
        asrc = Wa[:, :DH].reshape(F, 1) * mask
        adst = Wa[:, DH:].reshape(F, 1) * mask
    else:
        W2 = Wfc[0]
        asrc = jnp.tile(Wa[0, :F][:, None], (1, 16))
        adst = jnp.tile(Wa[0, F:][:, None], (1, 16))
    W2p = W2[_PERM]
    Wzs = jnp.concatenate([W2p[:, _PERM], W2p @ asrc], axis=1)
    Wad = W2p @ adst
    return Wzs, Wad


_BLK = 1000
_GRID = N // _BLK


def _dense0_body(nf, wemb, bemb, wzs, wad, h_o, zs_o, ad_o):
    h = jnp.dot(nf[...], wemb[...], preferred_element_type=jnp.float32) + bemb[...]
    h_o[...] = h
    zs_o[...] = jnp.dot(h, wzs[...], preferred_element_type=jnp.float32)
    ad_o[...] = jnp.dot(h, wad[...], preferred_element_type=jnp.float32)


def _epilogue(acc_ref, snc_ref, rden_ref, hprev_ref):
    acc = acc_ref[0] + acc_ref[1]
    msg = acc[:, :F]
    den = jnp.dot(acc[:, F:], rden_ref[...],
                  preferred_element_type=jnp.float32) + 1e-9
    x = snc_ref[...] * msg / den
    return hprev_ref[...] + jnp.where(x > 0, x, jnp.exp(x) - 1.0)


def _mid_body(hprev, acc, snc, rden, wzs, wad, h_o, zs_o, ad_o):
    h = _epilogue(acc, snc, rden, hprev)
    h_o[...] = h
    zs_o[...] = jnp.dot(h, wzs[...], preferred_element_type=jnp.float32)
    ad_o[...] = jnp.dot(h, wad[...], preferred_element_type=jnp.float32)


def _final_body(hprev, acc, snc, rden, w0, b0, w1, b1, w2, b2, out_o):
    h = _epilogue(acc, snc, rden, hprev)
    hg = jnp.mean(h, axis=0, keepdims=True)
    y = jnp.maximum(jnp.dot(hg, w0[...], preferred_element_type=jnp.float32)
                    + b0[...], 0.0)
    y = jnp.maximum(jnp.dot(y, w1[...], preferred_element_type=jnp.float32)
                    + b1[...], 0.0)
    lg = jnp.dot(y, w2[...], preferred_element_type=jnp.float32) + b2[...]
    out_o[...] = jnp.broadcast_to(lg, (8, F))


def _wspec(shape):
    return pl.BlockSpec(shape, lambda i: (0,) * len(shape))


_dense0 = pl.pallas_call(
    _dense0_body,
    grid=(_GRID,),
    in_specs=[pl.BlockSpec((_BLK, F), lambda i: (i, 0)),
              _wspec((F, F)), _wspec((1, F)),
              _wspec((F, ZS_W)), _wspec((F, 16))],
    out_specs=[pl.BlockSpec((_BLK, F), lambda i: (i, 0)),
               pl.BlockSpec((_BLK, ZS_W), lambda i: (i, 0)),
               pl.BlockSpec((_BLK, 16), lambda i: (i, 0))],
    out_shape=[jax.ShapeDtypeStruct((N, F), jnp.float32),
               jax.ShapeDtypeStruct((N, ZS_W), jnp.float32),
               jax.ShapeDtypeStruct((N, 16), jnp.float32)],
)

_mid = pl.pallas_call(
    _mid_body,
    grid=(_GRID,),
    in_specs=[pl.BlockSpec((_BLK, F), lambda i: (i, 0)),
              pl.BlockSpec((NC, _BLK, ZS_W), lambda i: (0, i, 0)),
              pl.BlockSpec((_BLK, 1), lambda i: (i, 0)),
              _wspec((16, F)),
              _wspec((F, ZS_W)), _wspec((F, 16))],
    out_specs=[pl.BlockSpec((_BLK, F), lambda i: (i, 0)),
               pl.BlockSpec((_BLK, ZS_W), lambda i: (i, 0)),
               pl.BlockSpec((_BLK, 16), lambda i: (i, 0))],
    out_shape=[jax.ShapeDtypeStruct((N, F), jnp.float32),
               jax.ShapeDtypeStruct((N, ZS_W), jnp.float32),
               jax.ShapeDtypeStruct((N, 16), jnp.float32)],
)

_final = pl.pallas_call(
    _final_body,
    grid=(1,),
    in_specs=[_wspec((N, F)),
              _wspec((NC, N, ZS_W)),
              _wspec((N, 1)),
              _wspec((16, F)),
              _wspec((F, F)), _wspec((1, F)),
              _wspec((F, F)), _wspec((1, F)),
              _wspec((F, F)), _wspec((1, F))],
    out_specs=pl.BlockSpec((8, F), lambda i: (0, 0)),
    out_shape=jax.ShapeDtypeStruct((8, F), jnp.float32),
)


@functools.cache
def _make_edge_sweep():
    mesh = plsc.VectorSubcoreMesh(core_axis_name="c", subcore_axis_name="s",
                                  num_cores=NC, num_subcores=NS)
    return functools.partial(
        pl.kernel,
        out_type=jax.ShapeDtypeStruct((NC, N_PAD, ZS_W), jnp.float32),
        mesh=mesh,
        compiler_params=pltpu.CompilerParams(use_tc_tiling_on_sc=False),
        scratch_types=[
            pltpu.VMEM((K,), jnp.int32),
            pltpu.VMEM((K,), jnp.int32),
            pltpu.VMEM((K, ZS_W), jnp.float32),
            pltpu.VMEM((K, 16), jnp.float32),
            pltpu.VMEM_SHARED((N_PAD, ZS_W), jnp.float32),
            pltpu.SemaphoreType.DMA,
            pltpu.SemaphoreType.DMA,
        ],
    )(_edge_sweep_body)


def _edge_sweep_body(src_hbm, dst_hbm, zs_hbm, ad_hbm, out_hbm,
                     srcv, dstv, zsv, bv, acc_sh, sem1, sem2):
    c = lax.axis_index("c")
    s = lax.axis_index("s")
    zero16 = jnp.zeros((LANES,), jnp.float32)

    def zrow(i, carry):
        for j in range(ZS_W // LANES):
            zsv[i, pl.ds(j * LANES, LANES)] = zero16
        return carry

    lax.fori_loop(0, K, zrow, 0)
    base_r = s * RPT
    off_r = 0
    while off_r < RPT:
        sz = min(K, RPT - off_r)
        pltpu.sync_copy(zsv.at[pl.ds(0, sz)],
                        acc_sh.at[pl.ds(base_r + off_r, sz)])
        off_r += sz
    plsc.subcore_barrier()

    ebase = (c * NS + s) * EPW

    def chunk(i, carry):
        off = ebase + i * K
        pltpu.sync_copy(src_hbm.at[pl.ds(off, K)], srcv)
        pltpu.sync_copy(dst_hbm.at[pl.ds(off, K)], dstv)
        cp1 = pltpu.async_copy(zs_hbm.at[srcv], zsv, sem1)
        cp2 = pltpu.async_copy(ad_hbm.at[dstv], bv, sem2)
        cp1.wait()
        cp2.wait()

        def edge(e, ecarry):
            x = zsv[e, pl.ds(F, LANES)] + bv[e, pl.ds(0, LANES)]
            x = jnp.maximum(x, 0.2 * x)
            w = jnp.exp(x)
            zsv[e, pl.ds(F, LANES)] = w
            for k2 in range(F // LANES):
                zsv[e, pl.ds(k2 * LANES, LANES)] = (
                    zsv[e, pl.ds(k2 * LANES, LANES)] * w)
            return ecarry

        lax.fori_loop(0, K, edge, 0)
        pltpu.sync_copy(zsv, acc_sh.at[dstv], add=True)
        return carry

    lax.fori_loop(0, NCHUNK, chunk, 0)
    plsc.subcore_barrier()
    pltpu.sync_copy(acc_sh.at[pl.ds(base_r, RPT)],
                    out_hbm.at[c, pl.ds(base_r, RPT)])


def kernel(nodes_feat, edges_feat, nodes_num_norm_sqrt, edges_num_norm_sqrt,
           W_emb, b_emb, Wfc0, Wa0, Wfc1, Wa1, Wfc2, Wa2, Wfc3, Wa3,
           Wr0, br0, Wr1, br1, Wr2, br2, edge_index):
    del edges_feat, edges_num_norm_sqrt
    src = edge_index[0]
    dst = edge_index[1]
    snc = nodes_num_norm_sqrt[:, None]
    rden = jnp.asarray(_RDEN)

    wemb_p = W_emb[:, _PERM]
    bemb_p = b_emb[_PERM][None, :]
    layers = [_prep_layer(Wfc, Wa)
              for Wfc, Wa in ((Wfc0, Wa0), (Wfc1, Wa1), (Wfc2, Wa2), (Wfc3, Wa3))]

    w0 = jnp.zeros((F, F), jnp.float32).at[:, :64].set(Wr0[_PERM])
    b0 = jnp.zeros((1, F), jnp.float32).at[0, :64].set(br0)
    w1 = jnp.zeros((F, F), jnp.float32).at[:64, :32].set(Wr1)
    b1 = jnp.zeros((1, F), jnp.float32).at[0, :32].set(br1)
    w2 = jnp.zeros((F, F), jnp.float32).at[:32, :40].set(Wr2)
    b2 = jnp.zeros((1, F), jnp.float32).at[0, :40].set(br2)

    edge_sweep = _make_edge_sweep()
    h, zs, ad = _dense0(nodes_feat, wemb_p, bemb_p, *layers[0])
    for li in (1, 2, 3):
        acc = edge_sweep(src, dst, zs, ad)
        h, zs, ad = _mid(h, acc, snc, rden, *layers[li])
    acc = edge_sweep(src, dst, zs, ad)
    out = _final(h, acc, snc, rden, w0, b0, w1, b1, w2, b2)
    return out[0, :40]


# speedup vs baseline: 98.5312x; 1.3615x over previous
import functools

import jax
import jax.numpy as jnp
import numpy as np
from jax import lax
from jax.experimental import pallas as pl
from jax.experimental.pallas import tpu as pltpu
from jax.experimental.pallas import tpu_sc as plsc

N = 10000
E = 320000
F = 128
NH = 8
DH = 16
ZS_W = F + 16

NC, NS, LANES = 2, 16, 16
EPW = E // (NC * NS)
K = 200
NCHUNK = EPW // K
N_PAD = 10112
RPT = N_PAD // NS

_PERM = np.array([(l % 8) * 16 + 2 * k + l // 8
                  for k in range(8) for l in range(16)], dtype=np.int32)

_RDEN = np.zeros((16, F), dtype=np.float32)
for _l in range(8):
    _RDEN[_l, np.arange(F) % 8 == _l] = 1.0


def _prep_layer(Wfc, Wa):
    H = Wfc.shape[0]
    if H == NH:
        W2 = jnp.transpose(Wfc, (1, 0, 2)).reshape(F, F)
        h_of_d = np.arange(F) // DH
        mask = (h_of_d[:, None] == (np.arange(16)[None, :] % 8)).astype(np.float32)
        asrc = Wa[:, :DH].reshape(F, 1) * mask
        adst = Wa[:, DH:].reshape(F, 1) * mask
    else:
        W2 = Wfc[0]
        asrc = jnp.tile(Wa[0, :F][:, None], (1, 16))
        adst = jnp.tile(Wa[0, F:][:, None], (1, 16))
    W2p = W2[_PERM]
    Wzs = jnp.concatenate([W2p[:, _PERM], W2p @ asrc], axis=1)
    Wad = W2p @ adst
    return Wzs, Wad


_BLK = 1000
_GRID = N // _BLK


def _dense0_body(nf, wemb, bemb, wzs, wad, h_o, zs_o, ad_o):
    h = jnp.dot(nf[...], wemb[...], preferred_element_type=jnp.float32) + bemb[...]
    h_o[...] = h
    zs_o[...] = jnp.dot(h, wzs[...], preferred_element_type=jnp.float32)
    ad_o[...] = jnp.dot(h, wad[...], preferred_element_type=jnp.float32)


def _epilogue(acc_ref, snc_ref, rden_ref, hprev_ref):
    acc = acc_ref[0] + acc_ref[1]
    msg = acc[:, :F]
    den = jnp.dot(acc[:, F:], rden_ref[...],
                  preferred_element_type=jnp.float32) + 1e-9
    x = snc_ref[...] * msg / den
    return hprev_ref[...] + jnp.where(x > 0, x, jnp.exp(x) - 1.0)


def _mid_body(hprev, acc, snc, rden, wzs, wad, h_o, zs_o, ad_o):
    h = _epilogue(acc, snc, rden, hprev)
    h_o[...] = h
    zs_o[...] = jnp.dot(h, wzs[...], preferred_element_type=jnp.float32)
    ad_o[...] = jnp.dot(h, wad[...], preferred_element_type=jnp.float32)


def _final_body(hprev, acc, snc, rden, w0, b0, w1, b1, w2, b2, out_o):
    h = _epilogue(acc, snc, rden, hprev)
    hg = jnp.mean(h, axis=0, keepdims=True)
    y = jnp.maximum(jnp.dot(hg, w0[...], preferred_element_type=jnp.float32)
                    + b0[...], 0.0)
    y = jnp.maximum(jnp.dot(y, w1[...], preferred_element_type=jnp.float32)
                    + b1[...], 0.0)
    lg = jnp.dot(y, w2[...], preferred_element_type=jnp.float32) + b2[...]
    out_o[...] = jnp.broadcast_to(lg, (8, F))


def _wspec(shape):
    return pl.BlockSpec(shape, lambda i: (0,) * len(shape))


_dense0 = pl.pallas_call(
    _dense0_body,
    grid=(_GRID,),
    in_specs=[pl.BlockSpec((_BLK, F), lambda i: (i, 0)),
              _wspec((F, F)), _wspec((1, F)),
              _wspec((F, ZS_W)), _wspec((F, 16))],
    out_specs=[pl.BlockSpec((_BLK, F), lambda i: (i, 0)),
               pl.BlockSpec((_BLK, ZS_W), lambda i: (i, 0)),
               pl.BlockSpec((_BLK, 16), lambda i: (i, 0))],
    out_shape=[jax.ShapeDtypeStruct((N, F), jnp.float32),
               jax.ShapeDtypeStruct((N, ZS_W), jnp.float32),
               jax.ShapeDtypeStruct((N, 16), jnp.float32)],
)

_mid = pl.pallas_call(
    _mid_body,
    grid=(_GRID,),
    in_specs=[pl.BlockSpec((_BLK, F), lambda i: (i, 0)),
              pl.BlockSpec((NC, _BLK, ZS_W), lambda i: (0, i, 0)),
              pl.BlockSpec((_BLK, 1), lambda i: (i, 0)),
              _wspec((16, F)),
              _wspec((F, ZS_W)), _wspec((F, 16))],
    out_specs=[pl.BlockSpec((_BLK, F), lambda i: (i, 0)),
               pl.BlockSpec((_BLK, ZS_W), lambda i: (i, 0)),
               pl.BlockSpec((_BLK, 16), lambda i: (i, 0))],
    out_shape=[jax.ShapeDtypeStruct((N, F), jnp.float32),
               jax.ShapeDtypeStruct((N, ZS_W), jnp.float32),
               jax.ShapeDtypeStruct((N, 16), jnp.float32)],
)

_final = pl.pallas_call(
    _final_body,
    grid=(1,),
    in_specs=[_wspec((N, F)),
              _wspec((NC, N, ZS_W)),
              _wspec((N, 1)),
              _wspec((16, F)),
              _wspec((F, F)), _wspec((1, F)),
              _wspec((F, F)), _wspec((1, F)),
              _wspec((F, F)), _wspec((1, F))],
    out_specs=pl.BlockSpec((8, F), lambda i: (0, 0)),
    out_shape=jax.ShapeDtypeStruct((8, F), jnp.float32),
)


@functools.cache
def _make_edge_sweep():
    mesh = plsc.VectorSubcoreMesh(core_axis_name="c", subcore_axis_name="s",
                                  num_cores=NC, num_subcores=NS)
    return functools.partial(
        pl.kernel,
        out_type=jax.ShapeDtypeStruct((NC, N_PAD, ZS_W), jnp.float32),
        mesh=mesh,
        compiler_params=pltpu.CompilerParams(use_tc_tiling_on_sc=False),
        scratch_types=[
            pltpu.VMEM((K,), jnp.int32),
            pltpu.VMEM((K,), jnp.int32),
            pltpu.VMEM((K, ZS_W), jnp.float32),
            pltpu.VMEM((K, 16), jnp.float32),
            pltpu.VMEM_SHARED((N_PAD, ZS_W), jnp.float32),
            pltpu.SemaphoreType.DMA,
            pltpu.SemaphoreType.DMA,
        ],
    )(_edge_sweep_body)


def _edge_sweep_body(src_hbm, dst_hbm, zs_hbm, ad_hbm, out_hbm,
                     srcv, dstv, zsv, bv, acc_sh, sem1, sem2):
    c = lax.axis_index("c")
    s = lax.axis_index("s")
    zero16 = jnp.zeros((LANES,), jnp.float32)

    @plsc.parallel_loop(0, K, unroll=2)
    def _zrow(i):
        for j in range(ZS_W // LANES):
            zsv[i, pl.ds(j * LANES, LANES)] = zero16
    base_r = s * RPT
    off_r = 0
    while off_r < RPT:
        sz = min(K, RPT - off_r)
        pltpu.sync_copy(zsv.at[pl.ds(0, sz)],
                        acc_sh.at[pl.ds(base_r + off_r, sz)])
        off_r += sz
    plsc.subcore_barrier()

    ebase = (c * NS + s) * EPW

    def chunk(i, carry):
        off = ebase + i * K
        pltpu.sync_copy(src_hbm.at[pl.ds(off, K)], srcv)
        pltpu.sync_copy(dst_hbm.at[pl.ds(off, K)], dstv)
        cp1 = pltpu.async_copy(zs_hbm.at[srcv], zsv, sem1)
        cp2 = pltpu.async_copy(ad_hbm.at[dstv], bv, sem2)
        cp1.wait()
        cp2.wait()

        @plsc.parallel_loop(0, K, unroll=4)
        def _edge(e):
            x = zsv[e, pl.ds(F, LANES)] + bv[e, pl.ds(0, LANES)]
            x = jnp.maximum(x, 0.2 * x)
            w = jnp.exp(x)
            zsv[e, pl.ds(F, LANES)] = w
            for k2 in range(F // LANES):
                zsv[e, pl.ds(k2 * LANES, LANES)] = (
                    zsv[e, pl.ds(k2 * LANES, LANES)] * w)
        pltpu.sync_copy(zsv, acc_sh.at[dstv], add=True)
        return carry

    lax.fori_loop(0, NCHUNK, chunk, 0)
    plsc.subcore_barrier()
    pltpu.sync_copy(acc_sh.at[pl.ds(base_r, RPT)],
                    out_hbm.at[c, pl.ds(base_r, RPT)])


def kernel(nodes_feat, edges_feat, nodes_num_norm_sqrt, edges_num_norm_sqrt,
           W_emb, b_emb, Wfc0, Wa0, Wfc1, Wa1, Wfc2, Wa2, Wfc3, Wa3,
           Wr0, br0, Wr1, br1, Wr2, br2, edge_index):
    del edges_feat, edges_num_norm_sqrt
    src = edge_index[0]
    dst = edge_index[1]
    snc = nodes_num_norm_sqrt[:, None]
    rden = jnp.asarray(_RDEN)

    wemb_p = W_emb[:, _PERM]
    bemb_p = b_emb[_PERM][None, :]
    layers = [_prep_layer(Wfc, Wa)
              for Wfc, Wa in ((Wfc0, Wa0), (Wfc1, Wa1), (Wfc2, Wa2), (Wfc3, Wa3))]

    w0 = jnp.zeros((F, F), jnp.float32).at[:, :64].set(Wr0[_PERM])
    b0 = jnp.zeros((1, F), jnp.float32).at[0, :64].set(br0)
    w1 = jnp.zeros((F, F), jnp.float32).at[:64, :32].set(Wr1)
    b1 = jnp.zeros((1, F), jnp.float32).at[0, :32].set(br1)
    w2 = jnp.zeros((F, F), jnp.float32).at[:32, :40].set(Wr2)
    b2 = jnp.zeros((1, F), jnp.float32).at[0, :40].set(br2)

    edge_sweep = _make_edge_sweep()
    h, zs, ad = _dense0(nodes_feat, wemb_p, bemb_p, *layers[0])
    for li in (1, 2, 3):
        acc = edge_sweep(src, dst, zs, ad)
        h, zs, ad = _mid(h, acc, snc, rden, *layers[li])
    acc = edge_sweep(src, dst, zs, ad)
    out = _final(h, acc, snc, rden, w0, b0, w1, b1, w2, b2)
    return out[0, :40]


# speedup vs baseline: 162.2099x; 1.6463x over previous
import functools

import jax
import jax.numpy as jnp
import numpy as np
from jax import lax
from jax.experimental import pallas as pl
from jax.experimental.pallas import tpu as pltpu
from jax.experimental.pallas import tpu_sc as plsc

N = 10000
E = 320000
F = 128
NH = 8
DH = 16
ZS_W = F + 16

NC, NS, LANES = 2, 16, 16
EPW = E // (NC * NS)
K = 80
NCHUNK = EPW // K
N_PAD = 10112
RPT = N_PAD // NS

_PERM = np.array([(l % 8) * 16 + 2 * k + l // 8
                  for k in range(8) for l in range(16)], dtype=np.int32)

_RDEN = np.zeros((16, F), dtype=np.float32)
for _l in range(8):
    _RDEN[_l, np.arange(F) % 8 == _l] = 1.0


def _prep_layer(Wfc, Wa):
    H = Wfc.shape[0]
    if H == NH:
        W2 = jnp.transpose(Wfc, (1, 0, 2)).reshape(F, F)
        h_of_d = np.arange(F) // DH
        mask = (h_of_d[:, None] == (np.arange(16)[None, :] % 8)).astype(np.float32)
        asrc = Wa[:, :DH].reshape(F, 1) * mask
        adst = Wa[:, DH:].reshape(F, 1) * mask
    else:
        W2 = Wfc[0]
        asrc = jnp.tile(Wa[0, :F][:, None], (1, 16))
        adst = jnp.tile(Wa[0, F:][:, None], (1, 16))
    W2p = W2[_PERM]
    Wzs = jnp.concatenate([W2p[:, _PERM], W2p @ asrc], axis=1)
    Wad = W2p @ adst
    return Wzs, Wad


_BLK = 1000
_GRID = N // _BLK


def _dense0_body(nf, wemb, bemb, wzs, wad, h_o, zs_o, ad_o):
    h = jnp.dot(nf[...], wemb[...], preferred_element_type=jnp.float32) + bemb[...]
    h_o[...] = h
    zs_o[...] = jnp.dot(h, wzs[...], preferred_element_type=jnp.float32)
    ad_o[...] = jnp.dot(h, wad[...], preferred_element_type=jnp.float32)


def _epilogue(acc_ref, snc_ref, rden_ref, hprev_ref):
    acc = acc_ref[0] + acc_ref[1]
    msg = acc[:, :F]
    den = jnp.dot(acc[:, F:], rden_ref[...],
                  preferred_element_type=jnp.float32) + 1e-9
    x = snc_ref[...] * msg / den
    return hprev_ref[...] + jnp.where(x > 0, x, jnp.exp(x) - 1.0)


def _mid_body(hprev, acc, snc, rden, wzs, wad, h_o, zs_o, ad_o):
    h = _epilogue(acc, snc, rden, hprev)
    h_o[...] = h
    zs_o[...] = jnp.dot(h, wzs[...], preferred_element_type=jnp.float32)
    ad_o[...] = jnp.dot(h, wad[...], preferred_element_type=jnp.float32)


def _final_body(hprev, acc, snc, rden, w0, b0, w1, b1, w2, b2, out_o):
    h = _epilogue(acc, snc, rden, hprev)
    hg = jnp.mean(h, axis=0, keepdims=True)
    y = jnp.maximum(jnp.dot(hg, w0[...], preferred_element_type=jnp.float32)
                    + b0[...], 0.0)
    y = jnp.maximum(jnp.dot(y, w1[...], preferred_element_type=jnp.float32)
                    + b1[...], 0.0)
    lg = jnp.dot(y, w2[...], preferred_element_type=jnp.float32) + b2[...]
    out_o[...] = jnp.broadcast_to(lg, (8, F))


def _wspec(shape):
    return pl.BlockSpec(shape, lambda i: (0,) * len(shape))


_dense0 = pl.pallas_call(
    _dense0_body,
    grid=(_GRID,),
    in_specs=[pl.BlockSpec((_BLK, F), lambda i: (i, 0)),
              _wspec((F, F)), _wspec((1, F)),
              _wspec((F, ZS_W)), _wspec((F, 16))],
    out_specs=[pl.BlockSpec((_BLK, F), lambda i: (i, 0)),
               pl.BlockSpec((_BLK, ZS_W), lambda i: (i, 0)),
               pl.BlockSpec((_BLK, 16), lambda i: (i, 0))],
    out_shape=[jax.ShapeDtypeStruct((N, F), jnp.float32),
               jax.ShapeDtypeStruct((N, ZS_W), jnp.float32),
               jax.ShapeDtypeStruct((N, 16), jnp.float32)],
)

_mid = pl.pallas_call(
    _mid_body,
    grid=(_GRID,),
    in_specs=[pl.BlockSpec((_BLK, F), lambda i: (i, 0)),
              pl.BlockSpec((NC, _BLK, ZS_W), lambda i: (0, i, 0)),
              pl.BlockSpec((_BLK, 1), lambda i: (i, 0)),
              _wspec((16, F)),
              _wspec((F, ZS_W)), _wspec((F, 16))],
    out_specs=[pl.BlockSpec((_BLK, F), lambda i: (i, 0)),
               pl.BlockSpec((_BLK, ZS_W), lambda i: (i, 0)),
               pl.BlockSpec((_BLK, 16), lambda i: (i, 0))],
    out_shape=[jax.ShapeDtypeStruct((N, F), jnp.float32),
               jax.ShapeDtypeStruct((N, ZS_W), jnp.float32),
               jax.ShapeDtypeStruct((N, 16), jnp.float32)],
)

_final = pl.pallas_call(
    _final_body,
    grid=(1,),
    in_specs=[_wspec((N, F)),
              _wspec((NC, N, ZS_W)),
              _wspec((N, 1)),
              _wspec((16, F)),
              _wspec((F, F)), _wspec((1, F)),
              _wspec((F, F)), _wspec((1, F)),
              _wspec((F, F)), _wspec((1, F))],
    out_specs=pl.BlockSpec((8, F), lambda i: (0, 0)),
    out_shape=jax.ShapeDtypeStruct((8, F), jnp.float32),
)


NBUF = 3


@functools.cache
def _make_edge_sweep():
    mesh = plsc.VectorSubcoreMesh(core_axis_name="c", subcore_axis_name="s",
                                  num_cores=NC, num_subcores=NS)
    scratch = ([pltpu.VMEM((K,), jnp.int32) for _ in range(NBUF)]
               + [pltpu.VMEM((K,), jnp.int32) for _ in range(NBUF)]
               + [pltpu.VMEM((K,), jnp.int32) for _ in range(NBUF)]
               + [pltpu.VMEM((K, ZS_W), jnp.float32) for _ in range(NBUF)]
               + [pltpu.VMEM((K, 16), jnp.float32) for _ in range(NBUF)]
               + [pltpu.VMEM_SHARED((N_PAD, ZS_W), jnp.float32)]
               + [pltpu.SemaphoreType.DMA for _ in range(4 * NBUF)])
    return functools.partial(
        pl.kernel,
        out_type=jax.ShapeDtypeStruct((NC, N_PAD, ZS_W), jnp.float32),
        mesh=mesh,
        compiler_params=pltpu.CompilerParams(use_tc_tiling_on_sc=False),
        scratch_types=scratch,
    )(_edge_sweep_body)


def _edge_sweep_body(pk_hbm, zs_hbm, ad_hbm, out_hbm, *refs):
    ibuf = refs[0:NBUF]
    sv = refs[NBUF:2 * NBUF]
    dv = refs[2 * NBUF:3 * NBUF]
    zsv = refs[3 * NBUF:4 * NBUF]
    bv = refs[4 * NBUF:5 * NBUF]
    acc_sh = refs[5 * NBUF]
    sem_i = refs[5 * NBUF + 1:5 * NBUF + 1 + NBUF]
    sem_gz = refs[5 * NBUF + 1 + NBUF:5 * NBUF + 1 + 2 * NBUF]
    sem_ga = refs[5 * NBUF + 1 + 2 * NBUF:5 * NBUF + 1 + 3 * NBUF]
    sem_s = refs[5 * NBUF + 1 + 3 * NBUF:5 * NBUF + 1 + 4 * NBUF]

    c = lax.axis_index("c")
    s = lax.axis_index("s")
    zero16 = jnp.zeros((LANES,), jnp.float32)

    @plsc.parallel_loop(0, K, unroll=2)
    def _zrow(i):
        for j in range(ZS_W // LANES):
            zsv[0][i, pl.ds(j * LANES, LANES)] = zero16

    base_r = s * RPT
    off_r = 0
    while off_r < RPT:
        sz = min(K, RPT - off_r)
        pltpu.sync_copy(zsv[0].at[pl.ds(0, sz)],
                        acc_sh.at[pl.ds(base_r + off_r, sz)])
        off_r += sz
    plsc.subcore_barrier()

    ebase = (c * NS + s) * EPW

    def _issue_idx(b, ci):
        return pltpu.async_copy(pk_hbm.at[pl.ds(ebase + ci * K, K)],
                                ibuf[b], sem_i[b])

    def _unpack(b):
        for g in range(K // LANES):
            v = ibuf[b][pl.ds(g * LANES, LANES)]
            sv[b][pl.ds(g * LANES, LANES)] = v & jnp.int32(0xFFFF)
            dv[b][pl.ds(g * LANES, LANES)] = v >> jnp.int32(16)

    def _issue_gathers(b):
        pltpu.async_copy(zs_hbm.at[sv[b]], zsv[b], sem_gz[b])
        pltpu.async_copy(ad_hbm.at[dv[b]], bv[b], sem_ga[b])

    def _wait_gathers(b):
        pltpu.make_async_copy(zs_hbm.at[sv[b]], zsv[b], sem_gz[b]).wait()
        pltpu.make_async_copy(ad_hbm.at[dv[b]], bv[b], sem_ga[b]).wait()

    def _issue_scatter(b):
        pltpu.async_copy(zsv[b], acc_sh.at[dv[b]], sem_s[b], add=True)

    def _wait_scatter(b):
        pltpu.make_async_copy(zsv[b], acc_sh.at[dv[b]], sem_s[b]).wait()

    def _compute(b):
        zb, bb = zsv[b], bv[b]

        @plsc.parallel_loop(0, K, unroll=4)
        def _edge(e):
            x = zb[e, pl.ds(F, LANES)] + bb[e, pl.ds(0, LANES)]
            x = jnp.maximum(x, 0.2 * x)
            w = jnp.exp(x)
            zb[e, pl.ds(F, LANES)] = w
            for k2 in range(F // LANES):
                zb[e, pl.ds(k2 * LANES, LANES)] = (
                    zb[e, pl.ds(k2 * LANES, LANES)] * w)

    _issue_idx(0, 0)
    _issue_idx(1, 1)
    pltpu.make_async_copy(pk_hbm.at[pl.ds(ebase, K)], ibuf[0], sem_i[0]).wait()
    _unpack(0)
    _issue_gathers(0)

    def chunk(i, carry):
        for par in range(NBUF):
            @pl.when(i % NBUF == par)
            def _(par=par):
                p, r, t = par, (par + 1) % NBUF, (par + 2) % NBUF

                @pl.when(i < NCHUNK - 2)
                def _():
                    _issue_idx(t, i + 2)

                @pl.when(jnp.logical_and(i >= 2, i < NCHUNK - 1))
                def _():
                    _wait_scatter(r)

                @pl.when(i < NCHUNK - 1)
                def _():
                    pltpu.make_async_copy(
                        pk_hbm.at[pl.ds(ebase + (i + 1) * K, K)],
                        ibuf[r], sem_i[r]).wait()
                    _unpack(r)
                    _issue_gathers(r)

                _wait_gathers(p)
                _compute(p)
                _issue_scatter(p)
        return carry

    lax.fori_loop(0, NCHUNK, chunk, 0)
    for ci in (NCHUNK - 3, NCHUNK - 2, NCHUNK - 1):
        _wait_scatter(ci % NBUF)
    plsc.subcore_barrier()
    pltpu.sync_copy(acc_sh.at[pl.ds(base_r, RPT)],
                    out_hbm.at[c, pl.ds(base_r, RPT)])


def kernel(nodes_feat, edges_feat, nodes_num_norm_sqrt, edges_num_norm_sqrt,
           W_emb, b_emb, Wfc0, Wa0, Wfc1, Wa1, Wfc2, Wa2, Wfc3, Wa3,
           Wr0, br0, Wr1, br1, Wr2, br2, edge_index):
    del edges_feat, edges_num_norm_sqrt
    src = edge_index[0]
    dst = edge_index[1]
    packed = src | (dst << jnp.int32(16))
    snc = nodes_num_norm_sqrt[:, None]
    rden = jnp.asarray(_RDEN)

    wemb_p = W_emb[:, _PERM]
    bemb_p = b_emb[_PERM][None, :]
    layers = [_prep_layer(Wfc, Wa)
              for Wfc, Wa in ((Wfc0, Wa0), (Wfc1, Wa1), (Wfc2, Wa2), (Wfc3, Wa3))]

    w0 = jnp.zeros((F, F), jnp.float32).at[:, :64].set(Wr0[_PERM])
    b0 = jnp.zeros((1, F), jnp.float32).at[0, :64].set(br0)
    w1 = jnp.zeros((F, F), jnp.float32).at[:64, :32].set(Wr1)
    b1 = jnp.zeros((1, F), jnp.float32).at[0, :32].set(br1)
    w2 = jnp.zeros((F, F), jnp.float32).at[:32, :40].set(Wr2)
    b2 = jnp.zeros((1, F), jnp.float32).at[0, :40].set(br2)

    edge_sweep = _make_edge_sweep()
    h, zs, ad = _dense0(nodes_feat, wemb_p, bemb_p, *layers[0])
    for li in (1, 2, 3):
        acc = edge_sweep(packed, zs, ad)
        h, zs, ad = _mid(h, acc, snc, rden, *layers[li])
    acc = edge_sweep(packed, zs, ad)
    out = _final(h, acc, snc, rden, w0, b0, w1, b1, w2, b2)
    return out[0, :40]
